# baseline (speedup 1.0000x reference)
import functools

import jax
import jax.numpy as jnp
from jax import lax
from jax.experimental import pallas as pl
from jax.experimental.pallas import tpu as pltpu
from jax.experimental.pallas import tpu_sc as plsc

D = 1024
E = 64
K = 8
G = 8
KG = 4
F = 512
FS = 1024
T = 2048
P = T * K
BLK = 512
PAD = P + E * BLK
NBLK = PAD // BLK
TB = 256

NC, NS = 2, 16
NW = NC * NS


def _gate_body(x_ref, gwt_ref, w_ref, idx_ref):
    xb = x_ref[...]
    logits = jnp.dot(xb, gwt_ref[...], preferred_element_type=jnp.float32)
    m = jnp.max(logits, axis=-1, keepdims=True)
    ex = jnp.exp(logits - m)
    scores = ex / jnp.sum(ex, axis=-1, keepdims=True)

    lane64 = lax.broadcasted_iota(jnp.int32, (TB, E), 1)
    gid = lane64 // G
    neg = jnp.float32(-jnp.inf)

    gs_cols = []
    for g in range(G):
        gs_cols.append(jnp.max(jnp.where(gid == g, scores, neg), axis=-1,
                               keepdims=True))
    lane8 = lax.broadcasted_iota(jnp.int32, (TB, G), 1)
    gs = jnp.zeros((TB, G), jnp.float32)
    for g in range(G):
        gs = jnp.where(lane8 == g, gs_cols[g], gs)

    gmask = jnp.zeros((TB, G), jnp.float32)
    gm = gs
    big8 = jnp.int32(G + 1)
    for _ in range(KG):
        mx = jnp.max(gm, axis=-1, keepdims=True)
        sel = jnp.min(jnp.where(gm == mx, lane8, big8), axis=-1, keepdims=True)
        oh = lane8 == sel
        gmask = jnp.maximum(gmask, oh.astype(jnp.float32))
        gm = jnp.where(oh, neg, gm)

    emask = jnp.zeros((TB, E), jnp.float32)
    for g in range(G):
        emask = jnp.where(gid == g, gmask[:, g:g + 1], emask)

    masked = scores * emask
    big64 = jnp.int32(E + 1)
    wout = jnp.zeros((TB, K), jnp.float32)
    iout = jnp.zeros((TB, K), jnp.int32)
    lane_k = lax.broadcasted_iota(jnp.int32, (TB, K), 1)
    mm = masked
    for k in range(K):
        mx = jnp.max(mm, axis=-1, keepdims=True)
        sel = jnp.min(jnp.where(mm == mx, lane64, big64), axis=-1,
                      keepdims=True)
        oh = lane64 == sel
        wk = jnp.sum(jnp.where(oh, scores, 0.0), axis=-1, keepdims=True)
        mm = jnp.where(oh, neg, mm)
        wout = jnp.where(lane_k == k, wk, wout)
        iout = jnp.where(lane_k == k, sel, iout)
    w_ref[...] = wout
    idx_ref[...] = iout


def _gate(x, gate_w):
    gwt = gate_w.T
    return pl.pallas_call(
        _gate_body,
        grid=(T // TB,),
        in_specs=[
            pl.BlockSpec((TB, D), lambda i: (i, 0)),
            pl.BlockSpec((D, E), lambda i: (0, 0)),
        ],
        out_specs=[
            pl.BlockSpec((TB, K), lambda i: (i, 0)),
            pl.BlockSpec((TB, K), lambda i: (i, 0)),
        ],
        out_shape=[
            jax.ShapeDtypeStruct((T, K), jnp.float32),
            jax.ShapeDtypeStruct((T, K), jnp.int32),
        ],
    )(x, gwt)


CHUNK = 512
NCHUNK = P // CHUNK


def _slots_body(idx_ref, pos_ref, bexp_ref, nb_ref, nxt_ref, part_ref):
    r = lax.broadcasted_iota(jnp.int32, (CHUNK, CHUNK), 0)
    c = lax.broadcasted_iota(jnp.int32, (CHUNK, CHUNK), 1)
    L = (r > c).astype(jnp.float32)
    lane64c = lax.broadcasted_iota(jnp.int32, (CHUNK, E), 1)

    def pass1(ci, run):
        sl = pl.ds(ci * CHUNK, CHUNK)
        oh = (idx_ref[sl, :] == lane64c).astype(jnp.float32)
        rank = jnp.dot(L, oh, preferred_element_type=jnp.float32)
        pick_rank = jnp.sum(rank * oh, axis=-1, keepdims=True)
        pick_prior = jnp.sum(run * oh, axis=-1, keepdims=True)
        part_ref[sl, :] = pick_rank + pick_prior
        return run + jnp.sum(oh, axis=0, keepdims=True)

    counts = lax.fori_loop(0, NCHUNK, pass1,
                           jnp.zeros((1, E), jnp.float32))

    counts_i = counts.astype(jnp.int32)
    padded_i = (counts_i + (BLK - 1)) // BLK * BLK
    padded = padded_i.astype(jnp.float32)
    ru = lax.broadcasted_iota(jnp.int32, (E, E), 0)
    cu = lax.broadcasted_iota(jnp.int32, (E, E), 1)
    U = (ru < cu).astype(jnp.float32)
    starts = jnp.dot(padded, U, preferred_element_type=jnp.float32)

    def pass2(ci, carry):
        sl = pl.ds(ci * CHUNK, CHUNK)
        oh = (idx_ref[sl, :] == lane64c).astype(jnp.float32)
        pick_start = jnp.sum(starts * oh, axis=-1, keepdims=True)
        pos_ref[sl, :] = (part_ref[sl, :] + pick_start).astype(jnp.int32)
        return carry

    lax.fori_loop(0, NCHUNK, pass2, 0)

    ends = starts + padded
    srow = (lax.broadcasted_iota(jnp.int32, (NBLK, E), 0) * BLK
            ).astype(jnp.float32)
    cnt = jnp.sum((ends <= srow).astype(jnp.float32), axis=-1,
                  keepdims=True)
    bexp = jnp.minimum(cnt.astype(jnp.int32), E - 1)
    bexp_ref[...] = bexp
    total = jnp.sum(padded, axis=-1, keepdims=True)
    nb_ref[...] = (total / BLK).astype(jnp.int32)
    ohb = (bexp == lax.broadcasted_iota(jnp.int32, (NBLK, E), 1)
           ).astype(jnp.float32)
    nxt_start = jnp.sum(ohb * ends, axis=-1, keepdims=True)
    ncnt = jnp.sum((ends <= nxt_start).astype(jnp.float32), axis=-1,
                   keepdims=True)
    nxt = jnp.minimum(ncnt.astype(jnp.int32), E - 1)
    nxt_ref[...] = jnp.where(nxt_start < total, nxt, bexp)


def _slots(idx_col):
    return pl.pallas_call(
        _slots_body,
        out_shape=[
            jax.ShapeDtypeStruct((P, 1), jnp.int32),
            jax.ShapeDtypeStruct((NBLK, 1), jnp.int32),
            jax.ShapeDtypeStruct((1, 1), jnp.int32),
            jax.ShapeDtypeStruct((NBLK, 1), jnp.int32),
        ],
        scratch_shapes=[pltpu.VMEM((P, 1), jnp.float32)],
    )(idx_col)


CH3 = 32
PPW = P // NW
NCH3 = PPW // CH3


WREP = 128


def _dispatch_body(x_hbm, tok_hbm, slot_hbm, wrep_hbm, xs_hbm, ws_hbm,
                   tok_v0, tok_v1, slot_v0, slot_v1, rows_v0, rows_v1,
                   wrows_v0, wrows_v1, sem_g0, sem_g1, sem_s0, sem_s1):
    wid = lax.axis_index("s") * NC + lax.axis_index("c")
    base = wid * PPW
    tok_v = (tok_v0, tok_v1)
    slot_v = (slot_v0, slot_v1)
    rows_v = (rows_v0, rows_v1)
    wrows_v = (wrows_v0, wrows_v1)
    sem_g = (sem_g0, sem_g1)
    sem_s = (sem_s0, sem_s1)

    def load_idx(i, b):
        off = pl.multiple_of(base + i * CH3, CH3)
        pltpu.sync_copy(tok_hbm.at[pl.ds(off, CH3)], tok_v[b])
        pltpu.sync_copy(slot_hbm.at[pl.ds(off, CH3)], slot_v[b])
        pltpu.sync_copy(wrep_hbm.at[pl.ds(off, CH3)], wrows_v[b])

    def start_gather(b):
        return pltpu.async_copy(x_hbm.at[tok_v[b]], rows_v[b], sem_g[b])

    def start_scatter(b):
        return (pltpu.async_copy(rows_v[b], xs_hbm.at[slot_v[b]], sem_s[b]),
                pltpu.async_copy(wrows_v[b], ws_hbm.at[slot_v[b]], sem_s[b]))

    load_idx(0, 0)
    g = start_gather(0)
    sc_prev = None
    for i in range(NCH3):
        b = i % 2
        if i + 1 < NCH3:
            if sc_prev is not None:
                for d in sc_prev:
                    d.wait()
            load_idx(i + 1, 1 - b)
            g_next = start_gather(1 - b)
        g.wait()
        sc_cur = start_scatter(b)
        if i + 1 < NCH3:
            sc_prev, g = sc_cur, g_next
        else:
            if sc_prev is not None:
                for d in sc_prev:
                    d.wait()
            for d in sc_cur:
                d.wait()


@functools.cache
def _dispatch():
    return pl.kernel(
        _dispatch_body,
        out_type=(jax.ShapeDtypeStruct((PAD, D), jnp.float32),
                  jax.ShapeDtypeStruct((PAD, WREP), jnp.float32)),
        mesh=plsc.VectorSubcoreMesh(core_axis_name="c", subcore_axis_name="s",
                                    num_cores=NC, num_subcores=NS),
        scratch_types=[
            pltpu.VMEM((CH3,), jnp.int32),
            pltpu.VMEM((CH3,), jnp.int32),
            pltpu.VMEM((CH3,), jnp.int32),
            pltpu.VMEM((CH3,), jnp.int32),
            pltpu.VMEM((CH3, D), jnp.float32),
            pltpu.VMEM((CH3, D), jnp.float32),
            pltpu.VMEM((CH3, WREP), jnp.float32),
            pltpu.VMEM((CH3, WREP), jnp.float32),
            pltpu.SemaphoreType.DMA,
            pltpu.SemaphoreType.DMA,
            pltpu.SemaphoreType.DMA,
            pltpu.SemaphoreType.DMA,
        ],
    )


def _expert_body(bexp_ref, nb_ref, nxt_ref, xs_ref, ws_ref,
                 w1_hbm, w3_hbm, w2_hbm, ys_ref,
                 w1b, w3b, w2b, sems, slot_ref):
    b = pl.program_id(0)
    nb = nb_ref[0]
    bc = jnp.minimum(b, nb - 1)
    e = bexp_ref[bc]
    bf = jnp.bfloat16

    def fetch(eid, s):
        pltpu.make_async_copy(w1_hbm.at[eid], w1b.at[s], sems.at[s, 0]).start()
        pltpu.make_async_copy(w3_hbm.at[eid], w3b.at[s], sems.at[s, 1]).start()
        pltpu.make_async_copy(w2_hbm.at[eid], w2b.at[s], sems.at[s, 2]).start()

    def drain(s):
        pltpu.make_async_copy(w1_hbm.at[0], w1b.at[s], sems.at[s, 0]).wait()
        pltpu.make_async_copy(w3_hbm.at[0], w3b.at[s], sems.at[s, 1]).wait()
        pltpu.make_async_copy(w2_hbm.at[0], w2b.at[s], sems.at[s, 2]).wait()

    @pl.when(b == 0)
    def _():
        slot_ref[0] = 0
        fetch(e, 0)
        drain(0)

    change = jnp.logical_and(jnp.logical_and(b > 0, b < nb),
                             bexp_ref[jnp.maximum(b - 1, 0)]
                             != bexp_ref[jnp.minimum(b, NBLK - 1)])

    @pl.when(change)
    def _():
        s = slot_ref[0] ^ 1
        drain(s)
        slot_ref[0] = s

    nxt = nxt_ref[bc]
    seg_start = jnp.logical_or(b == 0, change)

    @pl.when(jnp.logical_and(seg_start, nxt != e))
    def _():
        fetch(nxt, slot_ref[0] ^ 1)

    @pl.when(b < nb)
    def _():
        s = slot_ref[0]
        xb = xs_ref[...].astype(bf)
        w1 = w1b[s].astype(bf)
        w3 = w3b[s].astype(bf)
        w2 = w2b[s].astype(bf)
        dn = (((1,), (1,)), ((), ()))
        a = lax.dot_general(xb, w1, dn, preferred_element_type=jnp.float32)
        bq = lax.dot_general(xb, w3, dn, preferred_element_type=jnp.float32)
        h = (a * jax.nn.sigmoid(a) * bq).astype(bf)
        y = lax.dot_general(h, w2, dn, preferred_element_type=jnp.float32)
        ys_ref[...] = y * ws_ref[:, 0:1]


def _experts(xs, ws, W1, W3, W2, bexp, nb, nxt):
    def clamp(b, nb_):
        return jnp.minimum(b, nb_[0] - 1)

    grid_spec = pltpu.PrefetchScalarGridSpec(
        num_scalar_prefetch=3,
        grid=(NBLK,),
        in_specs=[
            pl.BlockSpec((BLK, D),
                         lambda b, be, nb_, nx: (clamp(b, nb_), 0)),
            pl.BlockSpec((BLK, WREP),
                         lambda b, be, nb_, nx: (clamp(b, nb_), 0)),
            pl.BlockSpec(memory_space=pltpu.MemorySpace.HBM),
            pl.BlockSpec(memory_space=pltpu.MemorySpace.HBM),
            pl.BlockSpec(memory_space=pltpu.MemorySpace.HBM),
        ],
        out_specs=pl.BlockSpec((BLK, D),
                               lambda b, be, nb_, nx: (clamp(b, nb_), 0)),
        scratch_shapes=[
            pltpu.VMEM((2, F, D), jnp.float32),
            pltpu.VMEM((2, F, D), jnp.float32),
            pltpu.VMEM((2, D, F), jnp.float32),
            pltpu.SemaphoreType.DMA((2, 3)),
            pltpu.SMEM((1,), jnp.int32),
        ],
    )
    return pl.pallas_call(
        _expert_body,
        grid_spec=grid_spec,
        out_shape=jax.ShapeDtypeStruct((PAD, D), jnp.float32),
    )(bexp, nb, nxt, xs, ws, W1, W3, W2)


SB = 128


def _shared_body(x_ref, w1_ref, w3_ref, w2_ref, z_ref):
    bf = jnp.bfloat16
    xb = x_ref[...].astype(bf)
    dn = (((1,), (1,)), ((), ()))
    a = lax.dot_general(xb, w1_ref[...].astype(bf), dn,
                        preferred_element_type=jnp.float32)
    bq = lax.dot_general(xb, w3_ref[...].astype(bf), dn,
                         preferred_element_type=jnp.float32)
    h = (a * jax.nn.sigmoid(a) * bq).astype(bf)
    z_ref[...] = lax.dot_general(h, w2_ref[...].astype(bf), dn,
                                 preferred_element_type=jnp.float32)


def _shared(x, Ws1, Ws3, Ws2):
    return pl.pallas_call(
        _shared_body,
        grid=(T // SB,),
        in_specs=[
            pl.BlockSpec((SB, D), lambda i: (i, 0)),
            pl.BlockSpec((FS, D), lambda i: (0, 0)),
            pl.BlockSpec((FS, D), lambda i: (0, 0)),
            pl.BlockSpec((D, FS), lambda i: (0, 0)),
        ],
        out_specs=pl.BlockSpec((SB, D), lambda i: (i, 0)),
        out_shape=jax.ShapeDtypeStruct((T, D), jnp.float32),
    )(x, Ws1, Ws3, Ws2)


TPW = T // NW
TCH = 4
NCH5 = TPW // TCH
RCH = TCH * K
VL = 16


def _combine_body(ys_hbm, pos_hbm, z_hbm, y_hbm,
                  pos_v, rows_v0, rows_v1, z_v0, z_v1, out_v0, out_v1,
                  semg0, semg1, semz0, semz1, semo0, semo1):
    wid = lax.axis_index("s") * NC + lax.axis_index("c")
    tbase = wid * TPW
    pbase = pl.multiple_of(tbase * K, 8)
    pltpu.sync_copy(pos_hbm.at[pl.ds(pbase, TPW * K)], pos_v)
    rows_v = (rows_v0, rows_v1)
    z_v = (z_v0, z_v1)
    out_v = (out_v0, out_v1)
    semg = (semg0, semg1)
    semz = (semz0, semz1)
    semo = (semo0, semo1)

    def start_gather(ci, b):
        p0 = pl.multiple_of(ci * RCH, RCH)
        return pltpu.async_copy(ys_hbm.at[pos_v.at[pl.ds(p0, RCH)]],
                                rows_v[b], semg[b])

    def start_z(ci, b):
        t0 = tbase + ci * TCH
        return pltpu.async_copy(z_hbm.at[pl.ds(t0, TCH)], z_v[b], semz[b])

    def start_out(ci, b):
        t0 = tbase + ci * TCH
        return pltpu.async_copy(out_v[b], y_hbm.at[pl.ds(t0, TCH)], semo[b])

    g = start_gather(0, 0)
    z = start_z(0, 0)
    o_pend = {0: None, 1: None}
    for ci in range(NCH5):
        b = ci % 2
        if ci + 1 < NCH5:
            g_next = start_gather(ci + 1, 1 - b)
            z_next = start_z(ci + 1, 1 - b)
        g.wait()
        z.wait()
        if o_pend[b] is not None:
            o_pend[b].wait()
        for t in range(TCH):

            def feat(v, c_, _t=t, _b=b):
                sl = pl.ds(pl.multiple_of(v * VL, VL), VL)
                acc = z_v[_b][_t, sl]
                for k in range(K):
                    acc = acc + rows_v[_b][_t * K + k, sl]
                out_v[_b][_t, sl] = acc
                return c_

            lax.fori_loop(0, D // VL, feat, 0)
        o_pend[b] = start_out(ci, b)
        if ci + 1 < NCH5:
            g, z = g_next, z_next
    for b in (0, 1):
        if o_pend[b] is not None:
            o_pend[b].wait()


@functools.cache
def _combine():
    return pl.kernel(
        _combine_body,
        out_type=jax.ShapeDtypeStruct((T, D), jnp.float32),
        mesh=plsc.VectorSubcoreMesh(core_axis_name="c", subcore_axis_name="s",
                                    num_cores=NC, num_subcores=NS),
        scratch_types=[
            pltpu.VMEM((TPW * K,), jnp.int32),
            pltpu.VMEM((RCH, D), jnp.float32),
            pltpu.VMEM((RCH, D), jnp.float32),
            pltpu.VMEM((TCH, D), jnp.float32),
            pltpu.VMEM((TCH, D), jnp.float32),
            pltpu.VMEM((TCH, D), jnp.float32),
            pltpu.VMEM((TCH, D), jnp.float32),
            pltpu.SemaphoreType.DMA,
            pltpu.SemaphoreType.DMA,
            pltpu.SemaphoreType.DMA,
            pltpu.SemaphoreType.DMA,
            pltpu.SemaphoreType.DMA,
            pltpu.SemaphoreType.DMA,
        ],
    )


def kernel(x, gate_w, W1, W2, W3, Ws1, Ws2, Ws3):
    w8, idx8 = _gate(x, gate_w)
    pos_col, bexp, nb, nxt = _slots(idx8.reshape(P, 1))
    pos = pos_col.reshape(P)
    tok = jnp.repeat(jnp.arange(T, dtype=jnp.int32), K)
    wrep = jnp.broadcast_to(w8.reshape(P, 1), (P, WREP))
    xs, ws = _dispatch()(x, tok, pos, wrep)
    ys = _experts(xs, ws, W1, W3, W2, bexp.reshape(NBLK), nb.reshape(1),
                  nxt.reshape(NBLK))
    z = _shared(x, Ws1, Ws3, Ws2)
    return _combine()(ys, pos, z)


# speedup vs baseline: 1.0025x; 1.0025x over previous
import functools

import jax
import jax.numpy as jnp
from jax import lax
from jax.experimental import pallas as pl
from jax.experimental.pallas import tpu as pltpu
from jax.experimental.pallas import tpu_sc as plsc

D = 1024
E = 64
K = 8
G = 8
KG = 4
F = 512
FS = 1024
T = 2048
P = T * K
BLK = 256
PAD = P + E * BLK
NBLK = PAD // BLK
TB = 256

NC, NS = 2, 16
NW = NC * NS


def _gate_body(x_ref, gwt_ref, w_ref, idx_ref):
    xb = x_ref[...]
    logits = jnp.dot(xb, gwt_ref[...], preferred_element_type=jnp.float32)
    m = jnp.max(logits, axis=-1, keepdims=True)
    ex = jnp.exp(logits - m)
    scores = ex / jnp.sum(ex, axis=-1, keepdims=True)

    lane64 = lax.broadcasted_iota(jnp.int32, (TB, E), 1)
    gid = lane64 // G
    neg = jnp.float32(-jnp.inf)

    gs_cols = []
    for g in range(G):
        gs_cols.append(jnp.max(jnp.where(gid == g, scores, neg), axis=-1,
                               keepdims=True))
    lane8 = lax.broadcasted_iota(jnp.int32, (TB, G), 1)
    gs = jnp.zeros((TB, G), jnp.float32)
    for g in range(G):
        gs = jnp.where(lane8 == g, gs_cols[g], gs)

    gmask = jnp.zeros((TB, G), jnp.float32)
    gm = gs
    big8 = jnp.int32(G + 1)
    for _ in range(KG):
        mx = jnp.max(gm, axis=-1, keepdims=True)
        sel = jnp.min(jnp.where(gm == mx, lane8, big8), axis=-1, keepdims=True)
        oh = lane8 == sel
        gmask = jnp.maximum(gmask, oh.astype(jnp.float32))
        gm = jnp.where(oh, neg, gm)

    emask = jnp.zeros((TB, E), jnp.float32)
    for g in range(G):
        emask = jnp.where(gid == g, gmask[:, g:g + 1], emask)

    masked = scores * emask
    big64 = jnp.int32(E + 1)
    wout = jnp.zeros((TB, K), jnp.float32)
    iout = jnp.zeros((TB, K), jnp.int32)
    lane_k = lax.broadcasted_iota(jnp.int32, (TB, K), 1)
    mm = masked
    for k in range(K):
        mx = jnp.max(mm, axis=-1, keepdims=True)
        sel = jnp.min(jnp.where(mm == mx, lane64, big64), axis=-1,
                      keepdims=True)
        oh = lane64 == sel
        wk = jnp.sum(jnp.where(oh, scores, 0.0), axis=-1, keepdims=True)
        mm = jnp.where(oh, neg, mm)
        wout = jnp.where(lane_k == k, wk, wout)
        iout = jnp.where(lane_k == k, sel, iout)
    w_ref[...] = wout
    idx_ref[...] = iout


def _gate(x, gate_w):
    gwt = gate_w.T
    return pl.pallas_call(
        _gate_body,
        grid=(T // TB,),
        in_specs=[
            pl.BlockSpec((TB, D), lambda i: (i, 0)),
            pl.BlockSpec((D, E), lambda i: (0, 0)),
        ],
        out_specs=[
            pl.BlockSpec((TB, K), lambda i: (i, 0)),
            pl.BlockSpec((TB, K), lambda i: (i, 0)),
        ],
        out_shape=[
            jax.ShapeDtypeStruct((T, K), jnp.float32),
            jax.ShapeDtypeStruct((T, K), jnp.int32),
        ],
    )(x, gwt)


CHUNK = 512
NCHUNK = P // CHUNK


def _slots_body(idx_ref, pos_ref, bexp_ref, nb_ref, nxt_ref, part_ref):
    r = lax.broadcasted_iota(jnp.int32, (CHUNK, CHUNK), 0)
    c = lax.broadcasted_iota(jnp.int32, (CHUNK, CHUNK), 1)
    L = (r > c).astype(jnp.float32)
    lane64c = lax.broadcasted_iota(jnp.int32, (CHUNK, E), 1)

    def pass1(ci, run):
        sl = pl.ds(ci * CHUNK, CHUNK)
        oh = (idx_ref[sl, :] == lane64c).astype(jnp.float32)
        rank = jnp.dot(L, oh, preferred_element_type=jnp.float32)
        pick_rank = jnp.sum(rank * oh, axis=-1, keepdims=True)
        pick_prior = jnp.sum(run * oh, axis=-1, keepdims=True)
        part_ref[sl, :] = pick_rank + pick_prior
        return run + jnp.sum(oh, axis=0, keepdims=True)

    counts = lax.fori_loop(0, NCHUNK, pass1,
                           jnp.zeros((1, E), jnp.float32))

    counts_i = counts.astype(jnp.int32)
    padded_i = (counts_i + (BLK - 1)) // BLK * BLK
    padded = padded_i.astype(jnp.float32)
    ru = lax.broadcasted_iota(jnp.int32, (E, E), 0)
    cu = lax.broadcasted_iota(jnp.int32, (E, E), 1)
    U = (ru < cu).astype(jnp.float32)
    starts = jnp.dot(padded, U, preferred_element_type=jnp.float32)

    def pass2(ci, carry):
        sl = pl.ds(ci * CHUNK, CHUNK)
        oh = (idx_ref[sl, :] == lane64c).astype(jnp.float32)
        pick_start = jnp.sum(starts * oh, axis=-1, keepdims=True)
        pos_ref[sl, :] = (part_ref[sl, :] + pick_start).astype(jnp.int32)
        return carry

    lax.fori_loop(0, NCHUNK, pass2, 0)

    ends = starts + padded
    srow = (lax.broadcasted_iota(jnp.int32, (NBLK, E), 0) * BLK
            ).astype(jnp.float32)
    cnt = jnp.sum((ends <= srow).astype(jnp.float32), axis=-1,
                  keepdims=True)
    bexp = jnp.minimum(cnt.astype(jnp.int32), E - 1)
    bexp_ref[...] = bexp
    total = jnp.sum(padded, axis=-1, keepdims=True)
    nb_ref[...] = (total / BLK).astype(jnp.int32)
    ohb = (bexp == lax.broadcasted_iota(jnp.int32, (NBLK, E), 1)
           ).astype(jnp.float32)
    nxt_start = jnp.sum(ohb * ends, axis=-1, keepdims=True)
    ncnt = jnp.sum((ends <= nxt_start).astype(jnp.float32), axis=-1,
                   keepdims=True)
    nxt = jnp.minimum(ncnt.astype(jnp.int32), E - 1)
    nxt_ref[...] = jnp.where(nxt_start < total, nxt, bexp)


def _slots(idx_col):
    return pl.pallas_call(
        _slots_body,
        out_shape=[
            jax.ShapeDtypeStruct((P, 1), jnp.int32),
            jax.ShapeDtypeStruct((NBLK, 1), jnp.int32),
            jax.ShapeDtypeStruct((1, 1), jnp.int32),
            jax.ShapeDtypeStruct((NBLK, 1), jnp.int32),
        ],
        scratch_shapes=[pltpu.VMEM((P, 1), jnp.float32)],
    )(idx_col)


CH3 = 32
PPW = P // NW
NCH3 = PPW // CH3


WREP = 128


def _dispatch_body(x_hbm, tok_hbm, slot_hbm, wrep_hbm, xs_hbm, ws_hbm,
                   tok_v0, tok_v1, slot_v0, slot_v1, rows_v0, rows_v1,
                   wrows_v0, wrows_v1, sem_g0, sem_g1, sem_s0, sem_s1):
    wid = lax.axis_index("s") * NC + lax.axis_index("c")
    base = wid * PPW
    tok_v = (tok_v0, tok_v1)
    slot_v = (slot_v0, slot_v1)
    rows_v = (rows_v0, rows_v1)
    wrows_v = (wrows_v0, wrows_v1)
    sem_g = (sem_g0, sem_g1)
    sem_s = (sem_s0, sem_s1)

    def load_idx(i, b):
        off = pl.multiple_of(base + i * CH3, CH3)
        pltpu.sync_copy(tok_hbm.at[pl.ds(off, CH3)], tok_v[b])
        pltpu.sync_copy(slot_hbm.at[pl.ds(off, CH3)], slot_v[b])
        pltpu.sync_copy(wrep_hbm.at[pl.ds(off, CH3)], wrows_v[b])

    def start_gather(b):
        return pltpu.async_copy(x_hbm.at[tok_v[b]], rows_v[b], sem_g[b])

    def start_scatter(b):
        return (pltpu.async_copy(rows_v[b], xs_hbm.at[slot_v[b]], sem_s[b]),
                pltpu.async_copy(wrows_v[b], ws_hbm.at[slot_v[b]], sem_s[b]))

    load_idx(0, 0)
    g = start_gather(0)
    sc_prev = None
    for i in range(NCH3):
        b = i % 2
        if i + 1 < NCH3:
            if sc_prev is not None:
                for d in sc_prev:
                    d.wait()
            load_idx(i + 1, 1 - b)
            g_next = start_gather(1 - b)
        g.wait()
        sc_cur = start_scatter(b)
        if i + 1 < NCH3:
            sc_prev, g = sc_cur, g_next
        else:
            if sc_prev is not None:
                for d in sc_prev:
                    d.wait()
            for d in sc_cur:
                d.wait()


@functools.cache
def _dispatch():
    return pl.kernel(
        _dispatch_body,
        out_type=(jax.ShapeDtypeStruct((PAD, D), jnp.float32),
                  jax.ShapeDtypeStruct((PAD, WREP), jnp.float32)),
        mesh=plsc.VectorSubcoreMesh(core_axis_name="c", subcore_axis_name="s",
                                    num_cores=NC, num_subcores=NS),
        scratch_types=[
            pltpu.VMEM((CH3,), jnp.int32),
            pltpu.VMEM((CH3,), jnp.int32),
            pltpu.VMEM((CH3,), jnp.int32),
            pltpu.VMEM((CH3,), jnp.int32),
            pltpu.VMEM((CH3, D), jnp.float32),
            pltpu.VMEM((CH3, D), jnp.float32),
            pltpu.VMEM((CH3, WREP), jnp.float32),
            pltpu.VMEM((CH3, WREP), jnp.float32),
            pltpu.SemaphoreType.DMA,
            pltpu.SemaphoreType.DMA,
            pltpu.SemaphoreType.DMA,
            pltpu.SemaphoreType.DMA,
        ],
    )


def _expert_body(bexp_ref, nb_ref, nxt_ref, xs_ref, ws_ref,
                 w1_hbm, w3_hbm, w2_hbm, ys_ref,
                 w1b, w3b, w2b, sems, slot_ref):
    b = pl.program_id(0)
    nb = nb_ref[0]
    bc = jnp.minimum(b, nb - 1)
    e = bexp_ref[bc]
    bf = jnp.bfloat16

    def fetch(eid, s):
        pltpu.make_async_copy(w1_hbm.at[eid], w1b.at[s], sems.at[s, 0]).start()
        pltpu.make_async_copy(w3_hbm.at[eid], w3b.at[s], sems.at[s, 1]).start()
        pltpu.make_async_copy(w2_hbm.at[eid], w2b.at[s], sems.at[s, 2]).start()

    def drain(s):
        pltpu.make_async_copy(w1_hbm.at[0], w1b.at[s], sems.at[s, 0]).wait()
        pltpu.make_async_copy(w3_hbm.at[0], w3b.at[s], sems.at[s, 1]).wait()
        pltpu.make_async_copy(w2_hbm.at[0], w2b.at[s], sems.at[s, 2]).wait()

    @pl.when(b == 0)
    def _():
        slot_ref[0] = 0
        fetch(e, 0)
        drain(0)

    change = jnp.logical_and(jnp.logical_and(b > 0, b < nb),
                             bexp_ref[jnp.maximum(b - 1, 0)]
                             != bexp_ref[jnp.minimum(b, NBLK - 1)])

    @pl.when(change)
    def _():
        s = slot_ref[0] ^ 1
        drain(s)
        slot_ref[0] = s

    nxt = nxt_ref[bc]
    seg_start = jnp.logical_or(b == 0, change)

    @pl.when(jnp.logical_and(seg_start, nxt != e))
    def _():
        fetch(nxt, slot_ref[0] ^ 1)

    @pl.when(b < nb)
    def _():
        s = slot_ref[0]
        xb = xs_ref[...].astype(bf)
        w1 = w1b[s].astype(bf)
        w3 = w3b[s].astype(bf)
        w2 = w2b[s].astype(bf)
        dn = (((1,), (1,)), ((), ()))
        a = lax.dot_general(xb, w1, dn, preferred_element_type=jnp.float32)
        bq = lax.dot_general(xb, w3, dn, preferred_element_type=jnp.float32)
        h = (a * jax.nn.sigmoid(a) * bq).astype(bf)
        y = lax.dot_general(h, w2, dn, preferred_element_type=jnp.float32)
        ys_ref[...] = y * ws_ref[:, 0:1]


def _experts(xs, ws, W1, W3, W2, bexp, nb, nxt):
    def clamp(b, nb_):
        return jnp.minimum(b, nb_[0] - 1)

    grid_spec = pltpu.PrefetchScalarGridSpec(
        num_scalar_prefetch=3,
        grid=(NBLK,),
        in_specs=[
            pl.BlockSpec((BLK, D),
                         lambda b, be, nb_, nx: (clamp(b, nb_), 0)),
            pl.BlockSpec((BLK, WREP),
                         lambda b, be, nb_, nx: (clamp(b, nb_), 0)),
            pl.BlockSpec(memory_space=pltpu.MemorySpace.HBM),
            pl.BlockSpec(memory_space=pltpu.MemorySpace.HBM),
            pl.BlockSpec(memory_space=pltpu.MemorySpace.HBM),
        ],
        out_specs=pl.BlockSpec((BLK, D),
                               lambda b, be, nb_, nx: (clamp(b, nb_), 0)),
        scratch_shapes=[
            pltpu.VMEM((2, F, D), jnp.float32),
            pltpu.VMEM((2, F, D), jnp.float32),
            pltpu.VMEM((2, D, F), jnp.float32),
            pltpu.SemaphoreType.DMA((2, 3)),
            pltpu.SMEM((1,), jnp.int32),
        ],
    )
    return pl.pallas_call(
        _expert_body,
        grid_spec=grid_spec,
        out_shape=jax.ShapeDtypeStruct((PAD, D), jnp.float32),
    )(bexp, nb, nxt, xs, ws, W1, W3, W2)


SB = 128


def _shared_body(x_ref, w1_ref, w3_ref, w2_ref, z_ref):
    bf = jnp.bfloat16
    xb = x_ref[...].astype(bf)
    dn = (((1,), (1,)), ((), ()))
    a = lax.dot_general(xb, w1_ref[...].astype(bf), dn,
                        preferred_element_type=jnp.float32)
    bq = lax.dot_general(xb, w3_ref[...].astype(bf), dn,
                         preferred_element_type=jnp.float32)
    h = (a * jax.nn.sigmoid(a) * bq).astype(bf)
    z_ref[...] = lax.dot_general(h, w2_ref[...].astype(bf), dn,
                                 preferred_element_type=jnp.float32)


def _shared(x, Ws1, Ws3, Ws2):
    return pl.pallas_call(
        _shared_body,
        grid=(T // SB,),
        in_specs=[
            pl.BlockSpec((SB, D), lambda i: (i, 0)),
            pl.BlockSpec((FS, D), lambda i: (0, 0)),
            pl.BlockSpec((FS, D), lambda i: (0, 0)),
            pl.BlockSpec((D, FS), lambda i: (0, 0)),
        ],
        out_specs=pl.BlockSpec((SB, D), lambda i: (i, 0)),
        out_shape=jax.ShapeDtypeStruct((T, D), jnp.float32),
    )(x, Ws1, Ws3, Ws2)


TPW = T // NW
TCH = 4
NCH5 = TPW // TCH
RCH = TCH * K
VL = 16


def _combine_body(ys_hbm, pos_hbm, z_hbm, y_hbm,
                  pos_v, rows_v0, rows_v1, z_v0, z_v1, out_v0, out_v1,
                  semg0, semg1, semz0, semz1, semo0, semo1):
    wid = lax.axis_index("s") * NC + lax.axis_index("c")
    tbase = wid * TPW
    pbase = pl.multiple_of(tbase * K, 8)
    pltpu.sync_copy(pos_hbm.at[pl.ds(pbase, TPW * K)], pos_v)
    rows_v = (rows_v0, rows_v1)
    z_v = (z_v0, z_v1)
    out_v = (out_v0, out_v1)
    semg = (semg0, semg1)
    semz = (semz0, semz1)
    semo = (semo0, semo1)

    def start_gather(ci, b):
        p0 = pl.multiple_of(ci * RCH, RCH)
        return pltpu.async_copy(ys_hbm.at[pos_v.at[pl.ds(p0, RCH)]],
                                rows_v[b], semg[b])

    def start_z(ci, b):
        t0 = tbase + ci * TCH
        return pltpu.async_copy(z_hbm.at[pl.ds(t0, TCH)], z_v[b], semz[b])

    def start_out(ci, b):
        t0 = tbase + ci * TCH
        return pltpu.async_copy(out_v[b], y_hbm.at[pl.ds(t0, TCH)], semo[b])

    g = start_gather(0, 0)
    z = start_z(0, 0)
    o_pend = {0: None, 1: None}
    for ci in range(NCH5):
        b = ci % 2
        if ci + 1 < NCH5:
            g_next = start_gather(ci + 1, 1 - b)
            z_next = start_z(ci + 1, 1 - b)
        g.wait()
        z.wait()
        if o_pend[b] is not None:
            o_pend[b].wait()
        for t in range(TCH):

            def feat(v, c_, _t=t, _b=b):
                sl = pl.ds(pl.multiple_of(v * VL, VL), VL)
                acc = z_v[_b][_t, sl]
                for k in range(K):
                    acc = acc + rows_v[_b][_t * K + k, sl]
                out_v[_b][_t, sl] = acc
                return c_

            lax.fori_loop(0, D // VL, feat, 0)
        o_pend[b] = start_out(ci, b)
        if ci + 1 < NCH5:
            g, z = g_next, z_next
    for b in (0, 1):
        if o_pend[b] is not None:
            o_pend[b].wait()


@functools.cache
def _combine():
    return pl.kernel(
        _combine_body,
        out_type=jax.ShapeDtypeStruct((T, D), jnp.float32),
        mesh=plsc.VectorSubcoreMesh(core_axis_name="c", subcore_axis_name="s",
                                    num_cores=NC, num_subcores=NS),
        scratch_types=[
            pltpu.VMEM((TPW * K,), jnp.int32),
            pltpu.VMEM((RCH, D), jnp.float32),
            pltpu.VMEM((RCH, D), jnp.float32),
            pltpu.VMEM((TCH, D), jnp.float32),
            pltpu.VMEM((TCH, D), jnp.float32),
            pltpu.VMEM((TCH, D), jnp.float32),
            pltpu.VMEM((TCH, D), jnp.float32),
            pltpu.SemaphoreType.DMA,
            pltpu.SemaphoreType.DMA,
            pltpu.SemaphoreType.DMA,
            pltpu.SemaphoreType.DMA,
            pltpu.SemaphoreType.DMA,
            pltpu.SemaphoreType.DMA,
        ],
    )


def kernel(x, gate_w, W1, W2, W3, Ws1, Ws2, Ws3):
    w8, idx8 = _gate(x, gate_w)
    pos_col, bexp, nb, nxt = _slots(idx8.reshape(P, 1))
    pos = pos_col.reshape(P)
    tok = jnp.repeat(jnp.arange(T, dtype=jnp.int32), K)
    wrep = jnp.broadcast_to(w8.reshape(P, 1), (P, WREP))
    xs, ws = _dispatch()(x, tok, pos, wrep)
    ys = _experts(xs, ws, W1, W3, W2, bexp.reshape(NBLK), nb.reshape(1),
                  nxt.reshape(NBLK))
    z = _shared(x, Ws1, Ws3, Ws2)
    return _combine()(ys, pos, z)
